# baseline (speedup 1.0000x reference)
import numpy as np
import jax
import jax.numpy as jnp
from jax.experimental import pallas as pl
from jax.experimental.pallas import tpu as pltpu

G = 8


def _row_select(H):
    Ho = H // 2
    R = np.zeros((H, H), np.float32)
    for a in range(Ho):
        R[a, 2 * a] = 1.0
        R[Ho + a, 2 * a + 1] = 1.0
    return jnp.asarray(R)


def _col_select(Wp, Wh, B):
    E = np.zeros((Wp, 2 * B), np.float32)
    for q in range(2):
        for j in range(Wh):
            s = 2 * j + q
            if s < Wp:
                E[s, q * B + j] = 1.0
    return jnp.asarray(E)


def _col_select2(W, Wo, B):
    E = np.zeros((W, 2 * B), np.float32)
    for j in range(Wo + 1):
        s = 2 * j - 1
        if 0 <= s < W:
            E[s, j] = 1.0
    for j in range(Wo):
        E[2 * j, B + j] = 1.0
    return jnp.asarray(E)


def _pool2_select():
    S = np.zeros((G * 64, G * 64), np.float32)
    for g in range(G):
        for di in range(2):
            for dj in range(2):
                ph = di * 2 + dj
                for i in range(4):
                    for j in range(4):
                        src = (2 * i + di) * 8 + (2 * j + dj)
                        S[g * 64 + ph * 16 + i * 4 + j, g * 64 + src] = 1.0
    return jnp.asarray(S)


def _conv5_select():
    S = np.zeros((G * 36, G * 16), np.float32)
    for g in range(G):
        for kh in range(3):
            for kw in range(3):
                t = kh * 3 + kw
                for a in range(2):
                    for b in range(2):
                        r, c = 2 * a + kh - 1, 2 * b + kw - 1
                        if 0 <= r < 4 and 0 <= c < 4:
                            S[g * 36 + t * 4 + a * 2 + b,
                              g * 16 + r * 4 + c] = 1.0
    return jnp.asarray(S)


def _conv3x3_s2_gemm(h, w, b, R, E, C, H, W, Cout, B, flat_out=False,
                     out_dtype=jnp.float32):
    Ho, Wo = H // 2, W // 2
    dt = h.dtype
    rows = jnp.stack(
        [jnp.dot(R, h[c], preferred_element_type=jnp.float32).astype(dt)
         for c in range(C)], axis=0)
    cols = jnp.dot(rows.reshape(C * H, W), E,
                   preferred_element_type=jnp.float32).astype(dt)
    cols = cols.reshape(C, 2, Ho, 2 * B)
    a_even = cols[:, 0]
    a_odd = cols[:, 1]
    a_oddm = jnp.concatenate(
        [jnp.zeros((C, 1, 2 * B), dt), a_odd[:, :Ho - 1, :]], axis=1)
    bases = (a_oddm, a_even, a_odd)
    taps = []
    for kh in range(3):
        for kw in range(3):
            q, j0 = ((0, 0), (1, 0), (0, 1))[kw]
            win = bases[kh][:, :, q * B + j0:q * B + j0 + Wo]
            taps.append(win.reshape(C, Ho * Wo))
    im = jnp.concatenate(taps, axis=0)
    out = (jnp.dot(w, im, preferred_element_type=jnp.float32)
           + b).astype(out_dtype)
    return out if flat_out else out.reshape(Cout, Ho, Wo)


def _maxpool2x2_gemm(h, Rp, Ep, C, H, W):
    Ho, Wo = H // 2, W // 2
    dt = h.dtype
    rows = jnp.stack(
        [jnp.dot(Rp, h[c], preferred_element_type=jnp.float32).astype(dt)
         for c in range(C)], axis=0)
    m = jnp.maximum(rows[:, :Ho, :], rows[:, Ho:, :])
    cols = jnp.dot(m.reshape(C * Ho, W), Ep,
                   preferred_element_type=jnp.float32).astype(dt)
    out = jnp.maximum(cols[:, :Wo], cols[:, 128:128 + Wo])
    return out.reshape(C, Ho, Wo)


def _body(x_ref, w0, b0, w1, b1, w2, b2, w3, b3,
          w5p, b5r, w6p, b6r, f0w, f0b, f1w, f1b, f2w, f2b,
          e0, e1, e2, e3, ep0, r0, r1, r2, r3, rp0, sp2, s5, o_ref):
    h = (x_ref[...].reshape(G * 3, 256, 256)
         .astype(jnp.bfloat16))
    h = _conv3x3_s2_gemm(h, w0[...], b0[...], r0[...], e0[...],
                         G * 3, 256, 256, G * 8, 256,
                         out_dtype=jnp.bfloat16)
    h = _conv3x3_s2_gemm(h, w1[...], b1[...], r1[...], e1[...],
                         G * 8, 128, 128, G * 16, 128,
                         out_dtype=jnp.bfloat16)
    h = _maxpool2x2_gemm(h, rp0[...], ep0[...], G * 16, 64, 64)
    h = _conv3x3_s2_gemm(h, w2[...], b2[...], r2[...], e2[...],
                         G * 16, 32, 32, G * 16, 128,
                         out_dtype=jnp.bfloat16)
    h = _conv3x3_s2_gemm(h, w3[...], b3[...], r3[...], e3[...],
                         G * 16, 16, 16, G * 64, 128,
                         flat_out=True)
    t = jnp.swapaxes(h.reshape(G, 64, 64), 1, 2)
    t = t.reshape(G * 64, 64)
    p2 = jnp.dot(sp2[...], t, preferred_element_type=jnp.float32)
    p2 = jnp.max(p2.reshape(G, 4, 16, 64), axis=1)
    p2 = p2.reshape(G * 16, 64)
    u = jnp.dot(s5[...], p2, preferred_element_type=jnp.float32)
    u = jnp.swapaxes(u.reshape(G, 9, 4, 64), 1, 2)
    u = u.reshape(G * 4, 576)
    v = (jnp.dot(u, w5p[...], preferred_element_type=jnp.float32)
         + b5r[...])
    v = v.reshape(G, 512)
    v = (jnp.dot(v, w6p[...], preferred_element_type=jnp.float32)
         + b6r[...])
    v = jnp.dot(v, f0w[...], preferred_element_type=jnp.float32) + f0b[...]
    v = jnp.dot(v, f1w[...], preferred_element_type=jnp.float32) + f1b[...]
    v = jnp.dot(v, f2w[...], preferred_element_type=jnp.float32) + f2b[...]
    o_ref[...] = v.astype(o_ref.dtype)


def _conv_w(w):
    cout, cin = w.shape[0], w.shape[1]
    return jnp.transpose(w, (2, 3, 1, 0)).reshape(9 * cin, cout).T


def _stack_w(w):
    cout, cin = w.shape[0], w.shape[1]
    wt = jnp.transpose(w, (0, 2, 3, 1)).reshape(cout, 9, cin)
    eye = jnp.eye(G, dtype=w.dtype)
    prod = eye[:, None, None, :, None] * wt[None, :, :, None, :]
    return prod.reshape(G * cout, 9 * G * cin)


def _stack_b(b):
    return jnp.tile(b, G).reshape(-1, 1)


def _full_spec(a):
    nd = a.ndim
    return pl.BlockSpec(a.shape, lambda n, nd=nd: (0,) * nd)


def kernel(x, cw0_w, cw0_b, cw1_w, cw1_b, cw2_w, cw2_b, cw3_w, cw3_b,
           cw4_w, cw4_b, cw5_w, cw5_b, fc0_w, fc0_b, fc1_w, fc1_b,
           fc2_w, fc2_b):
    N = x.shape[0]
    bf = jnp.bfloat16
    args = []
    for i, (w, b) in enumerate(((cw0_w, cw0_b), (cw1_w, cw1_b),
                                (cw2_w, cw2_b), (cw3_w, cw3_b))):
        args += [_stack_w(w).astype(bf), _stack_b(b)]
    args += [jnp.transpose(cw4_w, (2, 3, 1, 0)).reshape(576, 128),
             cw4_b.reshape(1, 128),
             jnp.transpose(cw5_w[:, :, 1:, 1:], (2, 3, 1, 0)).reshape(512, 128),
             cw5_b.reshape(1, 128)]
    for w, b in ((fc0_w, fc0_b), (fc1_w, fc1_b), (fc2_w, fc2_b)):
        args += [w.T, b.reshape(1, -1)]
    args += [_col_select2(256, 128, 256).astype(bf),
             _col_select2(128, 64, 128).astype(bf),
             _col_select2(32, 16, 128).astype(bf),
             _col_select2(16, 8, 128).astype(bf),
             _col_select(64, 32, 128).astype(bf),
             _row_select(256).astype(bf), _row_select(128).astype(bf),
             _row_select(32).astype(bf), _row_select(16).astype(bf),
             _row_select(64).astype(bf),
             _pool2_select(), _conv5_select()]

    out = pl.pallas_call(
        _body,
        out_shape=jax.ShapeDtypeStruct((N // G, G, 2), x.dtype),
        grid=(N // G,),
        in_specs=([pl.BlockSpec((G, 3, 256, 256), lambda n: (n, 0, 0, 0))]
                  + [_full_spec(a) for a in args]),
        out_specs=pl.BlockSpec((None, G, 2), lambda n: (n, 0, 0)),
        compiler_params=pltpu.CompilerParams(
            dimension_semantics=("parallel",)),
    )(x, *args)
    return out.reshape(N, 2)


# speedup vs baseline: 1.0214x; 1.0214x over previous
import numpy as np
import jax
import jax.numpy as jnp
from jax.experimental import pallas as pl
from jax.experimental.pallas import tpu as pltpu

G = 8


def _row_select(H):
    Ho = H // 2
    R = np.zeros((H, H), np.float32)
    for a in range(Ho):
        R[a, 2 * a] = 1.0
        R[Ho + a, 2 * a + 1] = 1.0
    return jnp.asarray(R)


def _col_select(Wp, Wh, B):
    E = np.zeros((Wp, 2 * B), np.float32)
    for q in range(2):
        for j in range(Wh):
            s = 2 * j + q
            if s < Wp:
                E[s, q * B + j] = 1.0
    return jnp.asarray(E)


def _col_select2(W, Wo, B):
    E = np.zeros((W, 2 * B), np.float32)
    for j in range(Wo + 1):
        s = 2 * j - 1
        if 0 <= s < W:
            E[s, j] = 1.0
    for j in range(Wo):
        E[2 * j, B + j] = 1.0
    return jnp.asarray(E)


def _pool2_select():
    S = np.zeros((G * 64, G * 64), np.float32)
    for g in range(G):
        for di in range(2):
            for dj in range(2):
                ph = di * 2 + dj
                for i in range(4):
                    for j in range(4):
                        src = (2 * i + di) * 8 + (2 * j + dj)
                        S[g * 64 + ph * 16 + i * 4 + j, g * 64 + src] = 1.0
    return jnp.asarray(S)


def _conv5_select():
    S = np.zeros((G * 36, G * 16), np.float32)
    for g in range(G):
        for kh in range(3):
            for kw in range(3):
                t = kh * 3 + kw
                for a in range(2):
                    for b in range(2):
                        r, c = 2 * a + kh - 1, 2 * b + kw - 1
                        if 0 <= r < 4 and 0 <= c < 4:
                            S[g * 36 + t * 4 + a * 2 + b,
                              g * 16 + r * 4 + c] = 1.0
    return jnp.asarray(S)


def _conv3x3_s2_gemm(h, w, b, R, E, C, H, W, Cout, B, flat_out=False,
                     out_dtype=jnp.float32):
    Ho, Wo = H // 2, W // 2
    dt = h.dtype
    rows = jnp.stack(
        [jnp.dot(R, h[c], preferred_element_type=jnp.float32).astype(dt)
         for c in range(C)], axis=0)
    cols = jnp.dot(rows.reshape(C * H, W), E,
                   preferred_element_type=jnp.float32).astype(dt)
    cols = cols.reshape(C, 2, Ho, 2 * B)
    a_even = cols[:, 0]
    a_odd = cols[:, 1]
    a_oddm = jnp.concatenate(
        [jnp.zeros((C, 1, 2 * B), dt), a_odd[:, :Ho - 1, :]], axis=1)
    bases = (a_oddm, a_even, a_odd)
    taps = []
    for kh in range(3):
        for kw in range(3):
            q, j0 = ((0, 0), (1, 0), (0, 1))[kw]
            win = bases[kh][:, :, q * B + j0:q * B + j0 + Wo]
            taps.append(win.reshape(C, Ho * Wo))
    im = jnp.concatenate(taps, axis=0)
    out = (jnp.dot(w, im, preferred_element_type=jnp.float32)
           + b).astype(out_dtype)
    return out if flat_out else out.reshape(Cout, Ho, Wo)


def _maxpool2x2_gemm(h, Rp, Ep, C, H, W):
    Ho, Wo = H // 2, W // 2
    dt = h.dtype
    rows = jnp.stack(
        [jnp.dot(Rp, h[c], preferred_element_type=jnp.float32).astype(dt)
         for c in range(C)], axis=0)
    m = jnp.maximum(rows[:, :Ho, :], rows[:, Ho:, :])
    cols = jnp.dot(m.reshape(C * Ho, W), Ep,
                   preferred_element_type=jnp.float32).astype(dt)
    out = jnp.maximum(cols[:, :Wo], cols[:, 128:128 + Wo])
    return out.reshape(C, Ho, Wo)


def _body(x_ref, w0, b0, w1, b1, w2, b2, w3, b3,
          w5p, b5r, w6p, b6r, f0w, f0b, f1w, f1b, f2w, f2b,
          e0, e1, e2, e3, ep0, r0, r1, r2, r3, rp0, sp2, s5, o_ref):
    h = (x_ref[...].reshape(G * 3, 256, 256)
         .astype(jnp.bfloat16))
    h = _conv3x3_s2_gemm(h, w0[...], b0[...], r0[...], e0[...],
                         G * 3, 256, 256, G * 8, 256,
                         out_dtype=jnp.bfloat16)
    h = _conv3x3_s2_gemm(h, w1[...], b1[...], r1[...], e1[...],
                         G * 8, 128, 128, G * 16, 128,
                         out_dtype=jnp.bfloat16)
    h = _maxpool2x2_gemm(h, rp0[...], ep0[...], G * 16, 64, 64)
    h = _conv3x3_s2_gemm(h, w2[...], b2[...], r2[...], e2[...],
                         G * 16, 32, 32, G * 16, 128,
                         out_dtype=jnp.bfloat16)
    h = _conv3x3_s2_gemm(h, w3[...], b3[...], r3[...], e3[...],
                         G * 16, 16, 16, G * 64, 128,
                         flat_out=True)
    t = jnp.swapaxes(h.reshape(G, 64, 64), 1, 2)
    t = t.reshape(G * 64, 64)
    p2 = jnp.dot(sp2[...], t, preferred_element_type=jnp.float32)
    p2 = jnp.max(p2.reshape(G, 4, 16, 64), axis=1)
    p2 = p2.reshape(G * 16, 64)
    u = jnp.dot(s5[...], p2, preferred_element_type=jnp.float32)
    u = jnp.swapaxes(u.reshape(G, 9, 4, 64), 1, 2)
    u = u.reshape(G * 4, 576)
    v = (jnp.dot(u, w5p[...], preferred_element_type=jnp.float32)
         + b5r[...])
    v = v.reshape(G, 512)
    v = (jnp.dot(v, w6p[...], preferred_element_type=jnp.float32)
         + b6r[...])
    v = jnp.dot(v, f0w[...], preferred_element_type=jnp.float32) + f0b[...]
    v = jnp.dot(v, f1w[...], preferred_element_type=jnp.float32) + f1b[...]
    v = jnp.dot(v, f2w[...], preferred_element_type=jnp.float32) + f2b[...]
    o_ref[...] = v.astype(o_ref.dtype)


def _conv_w(w):
    cout, cin = w.shape[0], w.shape[1]
    return jnp.transpose(w, (2, 3, 1, 0)).reshape(9 * cin, cout).T


def _stack_w(w):
    wt = _conv_w(w)
    cout, c9 = wt.shape
    c = c9 // 9
    k = jnp.kron(jnp.eye(G, dtype=w.dtype), wt)
    return (k.reshape(G * cout, G, 9, c).swapaxes(1, 2)
             .reshape(G * cout, 9 * G * c))


def _stack_b(b):
    return jnp.tile(b, G).reshape(-1, 1)


def _full_spec(a):
    nd = a.ndim
    return pl.BlockSpec(a.shape, lambda n, nd=nd: (0,) * nd)


def kernel(x, cw0_w, cw0_b, cw1_w, cw1_b, cw2_w, cw2_b, cw3_w, cw3_b,
           cw4_w, cw4_b, cw5_w, cw5_b, fc0_w, fc0_b, fc1_w, fc1_b,
           fc2_w, fc2_b):
    N = x.shape[0]
    bf = jnp.bfloat16
    args = []
    for i, (w, b) in enumerate(((cw0_w, cw0_b), (cw1_w, cw1_b),
                                (cw2_w, cw2_b), (cw3_w, cw3_b))):
        args += [_stack_w(w).astype(bf), _stack_b(b)]
    args += [jnp.transpose(cw4_w, (2, 3, 1, 0)).reshape(576, 128),
             cw4_b.reshape(1, 128),
             jnp.transpose(cw5_w[:, :, 1:, 1:], (2, 3, 1, 0)).reshape(512, 128),
             cw5_b.reshape(1, 128)]
    for w, b in ((fc0_w, fc0_b), (fc1_w, fc1_b), (fc2_w, fc2_b)):
        args += [w.T, b.reshape(1, -1)]
    args += [_col_select2(256, 128, 256).astype(bf),
             _col_select2(128, 64, 128).astype(bf),
             _col_select2(32, 16, 128).astype(bf),
             _col_select2(16, 8, 128).astype(bf),
             _col_select(64, 32, 128).astype(bf),
             _row_select(256).astype(bf), _row_select(128).astype(bf),
             _row_select(32).astype(bf), _row_select(16).astype(bf),
             _row_select(64).astype(bf),
             _pool2_select(), _conv5_select()]

    out = pl.pallas_call(
        _body,
        out_shape=jax.ShapeDtypeStruct((N // G, G, 2), x.dtype),
        grid=(N // G,),
        in_specs=([pl.BlockSpec((G, 3, 256, 256), lambda n: (n, 0, 0, 0))]
                  + [_full_spec(a) for a in args]),
        out_specs=pl.BlockSpec((None, G, 2), lambda n: (n, 0, 0)),
        compiler_params=pltpu.CompilerParams(
            dimension_semantics=("parallel",)),
    )(x, *args)
    return out.reshape(N, 2)
